# baseline (speedup 1.0000x reference)
import jax
import jax.numpy as jnp
from jax import lax
from jax.experimental import pallas as pl
from jax.experimental.pallas import tpu as pltpu
from jax.experimental.pallas import tpu_sc as plsc

_NC = 2
_NS = 16
_L = 16

_B, _T, _P, _H = 8, 4, 1025, 1280
_CH = 16
_NCH = 64
_JV = _H // _L


def _sc_body(hid, hlast, ids, table, gate16, out, out_last,
             ids_v, gate_v, rows16_v, row_v, last_v, in_bufs, out_bufs,
             row_sem, last_sem, in_sems, out_sems):
    wid = lax.axis_index("s") * _NC + lax.axis_index("c")
    b = wid // _T
    t = wid % _T

    pltpu.sync_copy(ids, ids_v.at[pl.ds(0, _B)])
    pltpu.sync_copy(gate16, gate_v)
    pltpu.async_copy(hlast.at[b, t], last_v, last_sem).wait()

    gv = gate_v[...]
    gt = 1.0 - 2.0 / (jnp.exp(2.0 * gv) + 1.0)

    bvec = jnp.full((_L,), b, dtype=jnp.int32)
    rid_vec = plsc.load_gather(ids_v, [bvec])
    idx_vec = rid_vec * _T + t
    pltpu.async_copy(table.at[idx_vec], rows16_v, row_sem).wait()

    for j in range(_JV):
        sl = pl.ds(j * _L, _L)
        row_v[sl] = rows16_v[0, sl] * gt

    def in_copy(off, n, k):
        return pltpu.make_async_copy(
            hid.at[b, t, pl.ds(off, n)], in_bufs.at[k, pl.ds(0, n)],
            in_sems.at[k])

    def out_copy(off, n, k):
        return pltpu.make_async_copy(
            out_bufs.at[k, pl.ds(0, n)], out.at[b, t, pl.ds(off, n)],
            out_sems.at[k])

    for j in range(_JV):
        sl = pl.ds(j * _L, _L)
        last_v[0, sl] = last_v[0, sl] + row_v[sl]
    pltpu.async_copy(last_v, out_last.at[b, t], last_sem).wait()

    def add_rows(k, nrows):
        def row_step(r, _):
            for j in range(_JV):
                sl = pl.ds(j * _L, _L)
                out_bufs[k, r, sl] = in_bufs[k, r, sl] + row_v[sl]
            return 0
        lax.fori_loop(0, nrows, row_step, 0)

    in_copy(0, _CH, 0).start()
    in_copy(_CH, _CH, 1).start()

    def chunk_step(c, _):
        k = lax.rem(c, 2)

        @pl.when(c >= 2)
        def _():
            out_copy((c - 2) * _CH, _CH, k).wait()

        in_copy(c * _CH, _CH, k).wait()
        add_rows(k, _CH)
        out_copy(c * _CH, _CH, k).start()

        @pl.when(c + 2 < _NCH)
        def _():
            in_copy((c + 2) * _CH, _CH, k).start()
        return 0

    lax.fori_loop(0, _NCH, chunk_step, 0)

    out_copy((_NCH - 2) * _CH, _CH, 0).wait()
    out_copy((_NCH - 1) * _CH, _CH, 1).wait()


@jax.jit
def _sc_call(hidden_state, hlast, ids32, table_rows, gate16):
    mesh = plsc.VectorSubcoreMesh(
        core_axis_name="c", subcore_axis_name="s",
        num_cores=_NC, num_subcores=_NS)
    fn = pl.kernel(
        _sc_body,
        out_type=(
            jax.ShapeDtypeStruct((_B, _T, _P, _H), jnp.float32),
            jax.ShapeDtypeStruct((_B, _T, 1, _H), jnp.float32),
        ),
        mesh=mesh,
        scratch_types=[
            pltpu.VMEM((2 * _L,), jnp.int32),
            pltpu.VMEM((_L,), jnp.float32),
            pltpu.VMEM((_L, _H), jnp.float32),
            pltpu.VMEM((_H,), jnp.float32),
            pltpu.VMEM((1, _H), jnp.float32),
            pltpu.VMEM((2, _CH, _H), jnp.float32),
            pltpu.VMEM((2, _CH, _H), jnp.float32),
            pltpu.SemaphoreType.DMA,
            pltpu.SemaphoreType.DMA,
            pltpu.SemaphoreType.DMA((2,)),
            pltpu.SemaphoreType.DMA((2,)),
        ],
        compiler_params=pltpu.CompilerParams(needs_layout_passes=False),
    )
    main, last = fn(hidden_state, hlast, ids32, table_rows, gate16)
    return lax.dynamic_update_slice(main, last, (0, 0, _P - 1, 0))


def kernel(hidden_state, aspect_ratio_ids, embedding_table, gate):
    B, T, P, H = hidden_state.shape
    V = embedding_table.shape[0]
    table_rows = embedding_table.reshape(V * T, H)
    ids32 = aspect_ratio_ids.astype(jnp.int32)
    gate16 = jnp.broadcast_to(gate.astype(jnp.float32), (_L,))
    hlast = lax.slice(hidden_state, (0, 0, P - 1, 0), (B, T, P, H))
    return _sc_call(hidden_state, hlast, ids32, table_rows, gate16)


# speedup vs baseline: 1.9558x; 1.9558x over previous
import jax
import jax.numpy as jnp
from jax import lax
from jax.experimental import pallas as pl
from jax.experimental.pallas import tpu as pltpu
from jax.experimental.pallas import tpu_sc as plsc

_NC = 2
_NS = 16
_L = 16

_B, _T, _P, _H = 8, 4, 1025, 1280
_CH = 16
_NCH = 64
_JV = _H // _L
_JG = 16


def _sc_body(hid, ids, table, gate16, out,
             ids_v, gate_v, rows16_v, row_v, last_v, in_bufs, out_bufs,
             row_sem, last_sem, in_sems, out_sems):
    wid = lax.axis_index("s") * _NC + lax.axis_index("c")
    b = wid // _T
    t = wid % _T

    pltpu.sync_copy(ids, ids_v.at[pl.ds(0, _B)])
    pltpu.sync_copy(gate16, gate_v)
    pltpu.async_copy(hid.at[b, t, pl.ds(_P - 1, 1)], last_v, last_sem).wait()

    gv = gate_v[...]
    gt = 1.0 - 2.0 / (jnp.exp(2.0 * gv) + 1.0)

    bvec = jnp.full((_L,), b, dtype=jnp.int32)
    rid_vec = plsc.load_gather(ids_v, [bvec])
    idx_vec = rid_vec * _T + t
    pltpu.async_copy(table.at[idx_vec], rows16_v, row_sem).wait()

    for j in range(_JV):
        sl = pl.ds(j * _L, _L)
        row_v[sl] = rows16_v[0, sl] * gt

    for j in range(_JV):
        sl = pl.ds(j * _L, _L)
        last_v[0, sl] = last_v[0, sl] + row_v[sl]
    pltpu.async_copy(last_v, out.at[b, t, pl.ds(_P - 1, 1)], last_sem).wait()

    def in_copy(off, k):
        return pltpu.make_async_copy(
            hid.at[b, t, pl.ds(off, _CH)], in_bufs.at[k], in_sems.at[k])

    def out_copy(off, k):
        return pltpu.make_async_copy(
            out_bufs.at[k], out.at[b, t, pl.ds(off, _CH)], out_sems.at[k])

    def add_rows(k):
        for jg in range(_JV // _JG):
            rv = [row_v[pl.ds((jg * _JG + u) * _L, _L)] for u in range(_JG)]

            def row_step(r, _):
                for u in range(_JG):
                    sl = pl.ds((jg * _JG + u) * _L, _L)
                    out_bufs[k, r, sl] = in_bufs[k, r, sl] + rv[u]
                return 0
            lax.fori_loop(0, _CH, row_step, 0)

    in_copy(0, 0).start()
    in_copy(_CH, 1).start()

    def chunk_step(c, _):
        k = lax.rem(c, 2)

        @pl.when(c >= 2)
        def _():
            out_copy((c - 2) * _CH, k).wait()

        in_copy(c * _CH, k).wait()
        add_rows(k)
        out_copy(c * _CH, k).start()

        @pl.when(c + 2 < _NCH)
        def _():
            in_copy((c + 2) * _CH, k).start()
        return 0

    lax.fori_loop(0, _NCH, chunk_step, 0)

    out_copy((_NCH - 2) * _CH, 0).wait()
    out_copy((_NCH - 1) * _CH, 1).wait()


@jax.jit
def _sc_call(hidden_state, ids32, table_rows, gate16):
    mesh = plsc.VectorSubcoreMesh(
        core_axis_name="c", subcore_axis_name="s",
        num_cores=_NC, num_subcores=_NS)
    fn = pl.kernel(
        _sc_body,
        out_type=jax.ShapeDtypeStruct((_B, _T, _P, _H), jnp.float32),
        mesh=mesh,
        scratch_types=[
            pltpu.VMEM((2 * _L,), jnp.int32),
            pltpu.VMEM((_L,), jnp.float32),
            pltpu.VMEM((_L, _H), jnp.float32),
            pltpu.VMEM((_H,), jnp.float32),
            pltpu.VMEM((1, _H), jnp.float32),
            pltpu.VMEM((2, _CH, _H), jnp.float32),
            pltpu.VMEM((2, _CH, _H), jnp.float32),
            pltpu.SemaphoreType.DMA,
            pltpu.SemaphoreType.DMA,
            pltpu.SemaphoreType.DMA((2,)),
            pltpu.SemaphoreType.DMA((2,)),
        ],
        compiler_params=pltpu.CompilerParams(needs_layout_passes=False),
    )
    return fn(hidden_state, ids32, table_rows, gate16)


def kernel(hidden_state, aspect_ratio_ids, embedding_table, gate):
    B, T, P, H = hidden_state.shape
    V = embedding_table.shape[0]
    table_rows = embedding_table.reshape(V * T, H)
    ids32 = aspect_ratio_ids.astype(jnp.int32)
    gate16 = jnp.broadcast_to(gate.astype(jnp.float32), (_L,))
    return _sc_call(hidden_state, ids32, table_rows, gate16)
